# baseline (speedup 1.0000x reference)
import functools

import jax
import jax.numpy as jnp
from jax import lax
from jax.experimental import pallas as pl
from jax.experimental.pallas import tpu as pltpu
from jax.experimental.pallas import tpu_sc as plsc

NC = 2
NS = 16
NW = NC * NS

DEG_W = 16


_C = 128


def _pick_rows_chunk(rows_per_tile):
    for c in range(128, 0, -1):
        if rows_per_tile % c == 0 and c % 8 == 0:
            return c
    raise ValueError("bad row count")


def _pad_rows(N):
    q = NS * 8
    return ((N + q - 1) // q) * q


def _strip_chunks(rpt):
    offs = []
    o = 0
    while o < rpt:
        k = min(_C, rpt - o)
        offs.append((o, k))
        o += k
    return offs


@functools.lru_cache(maxsize=None)
def _make_deg_kernel(n_chunks, N):
    C = _C
    Np = _pad_rows(N)
    rpt = Np // NS
    offs = _strip_chunks(rpt)
    mesh = plsc.VectorSubcoreMesh(core_axis_name="c", subcore_axis_name="s", num_cores=NC, num_subcores=NS)

    @functools.partial(
        pl.kernel,
        out_type=jax.ShapeDtypeStruct((NC, Np, DEG_W), jnp.float32),
        mesh=mesh,
        compiler_params=pltpu.CompilerParams(use_tc_tiling_on_sc=False),
        scratch_types=[
            pltpu.VMEM((n_chunks, C), jnp.int32),
            pltpu.VMEM((C, DEG_W), jnp.float32),
            pltpu.VMEM((C, DEG_W), jnp.float32),
            pltpu.VMEM_SHARED((Np, DEG_W), jnp.float32),
            pltpu.SemaphoreType.DMA,
        ],
    )
    def deg_kernel(dst_hbm, out_hbm, dst_v, ones_v, bounce_v, acc, sem):
        c = lax.axis_index("c")
        s = lax.axis_index("s")
        wid = c * NS + s

        pltpu.sync_copy(dst_hbm.at[wid], dst_v)

        def fill_ones(i, _):
            def fo(j, _):
                ones_v[i, pl.ds(j * 16, 16)] = jnp.ones((16,), jnp.float32)
                return 0
            lax.fori_loop(0, DEG_W // 16, fo, 0)
            return 0

        lax.fori_loop(0, C, fill_ones, 0)

        def fill_zero(i, _):
            def fz(j, _):
                bounce_v[i, pl.ds(j * 16, 16)] = jnp.zeros((16,), jnp.float32)
                return 0
            lax.fori_loop(0, DEG_W // 16, fz, 0)
            return 0

        lax.fori_loop(0, C, fill_zero, 0)

        for (o, k) in offs:
            pltpu.sync_copy(bounce_v.at[pl.ds(0, k)],
                            acc.at[pl.ds(s * rpt + o, k)])
        plsc.subcore_barrier()

        def chunk(i, _):
            pltpu.async_copy(ones_v, acc.at[dst_v.at[i]], sem, add=True)
            return 0

        lax.fori_loop(0, n_chunks, chunk, 0)

        def drain(i, _):
            pltpu.make_async_copy(ones_v, acc.at[dst_v.at[0]], sem).wait()
            return 0

        lax.fori_loop(0, n_chunks, drain, 0)
        plsc.subcore_barrier()

        for (o, k) in offs:
            pltpu.sync_copy(acc.at[pl.ds(s * rpt + o, k)],
                            bounce_v.at[pl.ds(0, k)])
            pltpu.sync_copy(bounce_v.at[pl.ds(0, k)],
                            out_hbm.at[c, pl.ds(s * rpt + o, k)])

    return deg_kernel


@functools.lru_cache(maxsize=None)
def _make_scatter_kernel(n_chunks, N, D):
    C = _C
    Np = _pad_rows(N)
    rpt = Np // NS
    offs = _strip_chunks(rpt)
    phases = 2 if n_chunks > 1 else 1
    npp = n_chunks // phases
    assert npp * phases == n_chunks and (npp % 2 == 0 or npp == 1)
    mesh = plsc.VectorSubcoreMesh(core_axis_name="c", subcore_axis_name="s", num_cores=NC, num_subcores=NS)

    @functools.partial(
        pl.kernel,
        out_type=jax.ShapeDtypeStruct((NC, Np, D), jnp.float32),
        mesh=mesh,
        scratch_types=[
            pltpu.VMEM((npp, C), jnp.int32),
            pltpu.VMEM((npp, C), jnp.int32),
            pltpu.VMEM((C, D), jnp.float32),
            pltpu.VMEM((C, D), jnp.float32),
            pltpu.VMEM_SHARED((Np, D), jnp.float32),
            pltpu.SemaphoreType.DMA,
            pltpu.SemaphoreType.DMA,
            pltpu.SemaphoreType.DMA,
            pltpu.SemaphoreType.DMA,
        ],
    )
    def scatter_kernel(g_hbm, src_hbm, dst_hbm, out_hbm,
                       src_v, dst_v, rows0, rows1, acc, sem0, sem1, ssem0, ssem1):
        c = lax.axis_index("c")
        s = lax.axis_index("s")
        wid = c * NS + s

        def fill_zero(i, _):
            def fz(j, _):
                rows0[i, pl.ds(j * 16, 16)] = jnp.zeros((16,), jnp.float32)
                return 0
            lax.fori_loop(0, D // 16, fz, 0)
            return 0

        lax.fori_loop(0, C, fill_zero, 0)

        for (o, k) in offs:
            pltpu.sync_copy(rows0.at[pl.ds(0, k)],
                            acc.at[pl.ds(s * rpt + o, k)])
        plsc.subcore_barrier()

        def gather_start(i, buf, sem):
            pltpu.async_copy(g_hbm.at[src_v.at[i]], buf, sem)

        def gather_wait(i, buf, sem):
            pltpu.make_async_copy(g_hbm.at[src_v.at[i]], buf, sem).wait()

        def scatter_start(i, buf, sem):
            pltpu.async_copy(buf, acc.at[dst_v.at[i]], sem, add=True)

        def scatter_wait(i, buf, sem):
            pltpu.make_async_copy(buf, acc.at[dst_v.at[i]], sem).wait()

        for p in range(phases):
            pltpu.sync_copy(src_hbm.at[wid, pl.ds(p * npp, npp)], src_v)
            pltpu.sync_copy(dst_hbm.at[wid, pl.ds(p * npp, npp)], dst_v)
            if npp == 1:
                gather_start(0, rows0, sem0)
                gather_wait(0, rows0, sem0)
                pltpu.sync_copy(rows0, acc.at[dst_v.at[0]], add=True)
                continue
            gather_start(0, rows0, sem0)
            gather_start(1, rows1, sem1)

            def body(k2, _):
                i0 = 2 * k2
                i1 = i0 + 1
                gather_wait(i0, rows0, sem0)
                scatter_start(i0, rows0, ssem0)
                gather_wait(i1, rows1, sem1)
                scatter_start(i1, rows1, ssem1)

                @pl.when(i0 + 2 < npp)
                def _():
                    scatter_wait(i0, rows0, ssem0)
                    gather_start(i0 + 2, rows0, sem0)

                @pl.when(i1 + 2 < npp)
                def _():
                    scatter_wait(i1, rows1, ssem1)
                    gather_start(i1 + 2, rows1, sem1)

                return 0

            lax.fori_loop(0, npp // 2, body, 0)
            scatter_wait(npp - 2, rows0, ssem0)
            scatter_wait(npp - 1, rows1, ssem1)
        plsc.subcore_barrier()

        for (o, k) in offs:
            pltpu.sync_copy(acc.at[pl.ds(s * rpt + o, k)],
                            rows0.at[pl.ds(0, k)])
            pltpu.sync_copy(rows0.at[pl.ds(0, k)],
                            out_hbm.at[c, pl.ds(s * rpt + o, k)])

    return scatter_kernel


_R = 2000


def _dinv_from_deg(deg_ref):
    deg = deg_ref[0, :, 0:1] + deg_ref[1, :, 0:1] + 1.0
    return lax.rsqrt(deg)


def _scale_mm_body(x_ref, w_ref, deg_ref, o_ref):
    o_ref[...] = jnp.dot(x_ref[...], w_ref[...],
                         preferred_element_type=jnp.float32) * _dinv_from_deg(deg_ref)


def _tc_scale_mm(x, w, degp):
    n, d = x.shape
    r = _R if n % _R == 0 else 8
    return pl.pallas_call(
        _scale_mm_body,
        grid=(n // r,),
        in_specs=[
            pl.BlockSpec((r, d), lambda i: (i, 0)),
            pl.BlockSpec((d, w.shape[1]), lambda i: (0, 0)),
            pl.BlockSpec((NC, r, DEG_W), lambda i: (0, i, 0)),
        ],
        out_specs=pl.BlockSpec((r, w.shape[1]), lambda i: (i, 0)),
        out_shape=jax.ShapeDtypeStruct((n, w.shape[1]), jnp.float32),
    )(x, w, degp)


def _layer_body(s_ref, g_ref, deg_ref, b_ref, w_ref, o_ref):
    dinv = _dinv_from_deg(deg_ref)
    tot = s_ref[0] + s_ref[1] + g_ref[...]
    h = jnp.maximum(dinv * tot + b_ref[...], 0.0)
    o_ref[...] = jnp.dot(h, w_ref[...],
                         preferred_element_type=jnp.float32) * dinv


def _tc_layer(S, g, degp, b, w):
    n, d = g.shape
    r = _R if n % _R == 0 else 8
    return pl.pallas_call(
        _layer_body,
        grid=(n // r,),
        in_specs=[
            pl.BlockSpec((NC, r, d), lambda i: (0, i, 0)),
            pl.BlockSpec((r, d), lambda i: (i, 0)),
            pl.BlockSpec((NC, r, DEG_W), lambda i: (0, i, 0)),
            pl.BlockSpec((1, d), lambda i: (0, 0)),
            pl.BlockSpec((d, d), lambda i: (0, 0)),
        ],
        out_specs=pl.BlockSpec((r, d), lambda i: (i, 0)),
        out_shape=jax.ShapeDtypeStruct((n, d), jnp.float32),
    )(S, g, degp, b, w)


def _final_body(s_ref, g_ref, deg_ref, b_ref, w_ref, b3_ref, o_ref):
    dinv = _dinv_from_deg(deg_ref)
    tot = s_ref[0] + s_ref[1] + g_ref[...]
    h = jnp.maximum(dinv * tot + b_ref[...], 0.0)
    res = jnp.dot(h, w_ref[...],
                  preferred_element_type=jnp.float32) + b3_ref[...]
    o_ref[...] = res[:, :o_ref.shape[1]]


def _tc_final(S, g, degp, b, w_pad, b3_pad, n_classes):
    n, d = g.shape
    r = _R if n % _R == 0 else 8
    dp = w_pad.shape[1]
    return pl.pallas_call(
        _final_body,
        grid=(n // r,),
        in_specs=[
            pl.BlockSpec((NC, r, d), lambda i: (0, i, 0)),
            pl.BlockSpec((r, d), lambda i: (i, 0)),
            pl.BlockSpec((NC, r, DEG_W), lambda i: (0, i, 0)),
            pl.BlockSpec((1, d), lambda i: (0, 0)),
            pl.BlockSpec((d, dp), lambda i: (0, 0)),
            pl.BlockSpec((1, dp), lambda i: (0, 0)),
        ],
        out_specs=pl.BlockSpec((r, n_classes), lambda i: (i, 0)),
        out_shape=jax.ShapeDtypeStruct((n, n_classes), jnp.float32),
    )(S, g, degp, b, w_pad, b3_pad)


def kernel(x, edge_index, W1, b1, W2, b2, W3, b3):
    n, d = x.shape
    e = edge_index.shape[1]

    n_classes = W3.shape[1]
    w3_pad = jnp.pad(W3.astype(jnp.float32), ((0, 0), (0, d - n_classes)))
    b3_pad = jnp.pad(b3.astype(jnp.float32), (0, d - n_classes)).reshape(1, d)
    b1r = b1.reshape(1, d)
    b2r = b2.reshape(1, d)

    quant = NW * _C
    n_chunks = -(-e // quant)
    if n_chunks > 1 and n_chunks % 4:
        n_chunks += 4 - n_chunks % 4
    e_pad = n_chunks * quant
    edges = edge_index.astype(jnp.int32)
    if e_pad != e:
        n_trash = _pad_rows(n) - n
        assert n_trash > 0, "no trash row available for edge padding"
        pad = e_pad - e
        pad_block = jnp.stack([jnp.arange(pad, dtype=jnp.int32) % n,
                               n + jnp.arange(pad, dtype=jnp.int32) % n_trash])
        edges = jnp.concatenate([edges, pad_block], axis=1)
    idx3 = edges.reshape(2, NW, n_chunks, _C)
    src3 = idx3[0]
    dst3 = idx3[1]

    deg_fn = _make_deg_kernel(n_chunks, n)
    scat_fn = _make_scatter_kernel(n_chunks, n, d)

    degp = deg_fn(dst3)
    g1 = _tc_scale_mm(x, W1, degp)
    S1 = scat_fn(g1, src3, dst3)
    g2 = _tc_layer(S1, g1, degp, b1r, W2)
    S2 = scat_fn(g2, src3, dst3)
    return _tc_final(S2, g2, degp, b2r, w3_pad, b3_pad, n_classes)


# speedup vs baseline: 1.2482x; 1.2482x over previous
import functools

import jax
import jax.numpy as jnp
from jax import lax
from jax.experimental import pallas as pl
from jax.experimental.pallas import tpu as pltpu
from jax.experimental.pallas import tpu_sc as plsc

NC = 2
NS = 16
NW = NC * NS

DEG_W = 16


_C = 128


def _pick_rows_chunk(rows_per_tile):
    for c in range(128, 0, -1):
        if rows_per_tile % c == 0 and c % 8 == 0:
            return c
    raise ValueError("bad row count")


def _pad_rows(N):
    q = NS * 8
    return ((N + q - 1) // q) * q


def _strip_chunks(rpt):
    offs = []
    o = 0
    while o < rpt:
        k = min(_C, rpt - o)
        offs.append((o, k))
        o += k
    return offs


@functools.lru_cache(maxsize=None)
def _make_deg_kernel(n_chunks, N):
    C = _C
    Np = _pad_rows(N)
    rpt = Np // NS
    offs = _strip_chunks(rpt)
    mesh = plsc.VectorSubcoreMesh(core_axis_name="c", subcore_axis_name="s", num_cores=NC, num_subcores=NS)

    @functools.partial(
        pl.kernel,
        out_type=jax.ShapeDtypeStruct((NC, Np, DEG_W), jnp.float32),
        mesh=mesh,
        compiler_params=pltpu.CompilerParams(use_tc_tiling_on_sc=False),
        scratch_types=[
            pltpu.VMEM((n_chunks, C), jnp.int32),
            pltpu.VMEM((C, DEG_W), jnp.float32),
            pltpu.VMEM((C, DEG_W), jnp.float32),
            pltpu.VMEM_SHARED((Np, DEG_W), jnp.float32),
            pltpu.SemaphoreType.DMA,
        ],
    )
    def deg_kernel(dst_hbm, out_hbm, dst_v, ones_v, bounce_v, acc, sem):
        c = lax.axis_index("c")
        s = lax.axis_index("s")
        wid = c * NS + s

        pltpu.sync_copy(dst_hbm.at[wid], dst_v)

        def fill_ones(i, _):
            def fo(j, _):
                ones_v[i, pl.ds(j * 16, 16)] = jnp.ones((16,), jnp.float32)
                return 0
            lax.fori_loop(0, DEG_W // 16, fo, 0)
            return 0

        lax.fori_loop(0, C, fill_ones, 0)

        def fill_zero(i, _):
            def fz(j, _):
                bounce_v[i, pl.ds(j * 16, 16)] = jnp.zeros((16,), jnp.float32)
                return 0
            lax.fori_loop(0, DEG_W // 16, fz, 0)
            return 0

        lax.fori_loop(0, C, fill_zero, 0)

        for (o, k) in offs:
            pltpu.sync_copy(bounce_v.at[pl.ds(0, k)],
                            acc.at[pl.ds(s * rpt + o, k)])
        plsc.subcore_barrier()

        def chunk(i, _):
            pltpu.async_copy(ones_v, acc.at[dst_v.at[i]], sem, add=True)
            return 0

        lax.fori_loop(0, n_chunks, chunk, 0)

        def drain(i, _):
            pltpu.make_async_copy(ones_v, acc.at[dst_v.at[0]], sem).wait()
            return 0

        lax.fori_loop(0, n_chunks, drain, 0)
        plsc.subcore_barrier()

        for (o, k) in offs:
            pltpu.sync_copy(acc.at[pl.ds(s * rpt + o, k)],
                            bounce_v.at[pl.ds(0, k)])
            pltpu.sync_copy(bounce_v.at[pl.ds(0, k)],
                            out_hbm.at[c, pl.ds(s * rpt + o, k)])

    return deg_kernel


@functools.lru_cache(maxsize=None)
def _make_scatter_kernel(n_chunks, N, D):
    C = _C
    Np = _pad_rows(N)
    rpt = Np // NS
    offs = _strip_chunks(rpt)
    phases = 2 if n_chunks > 1 else 1
    npp = n_chunks // phases
    assert npp * phases == n_chunks and (npp % 2 == 0 or npp == 1)
    mesh = plsc.VectorSubcoreMesh(core_axis_name="c", subcore_axis_name="s", num_cores=NC, num_subcores=NS)

    @functools.partial(
        pl.kernel,
        out_type=jax.ShapeDtypeStruct((NC, Np, D), jnp.float32),
        mesh=mesh,
        scratch_types=[
            pltpu.VMEM((npp, C), jnp.int32),
            pltpu.VMEM((npp, C), jnp.int32),
            pltpu.VMEM((C, D), jnp.float32),
            pltpu.VMEM((C, D), jnp.float32),
            pltpu.VMEM_SHARED((Np, D), jnp.float32),
            pltpu.SemaphoreType.DMA,
            pltpu.SemaphoreType.DMA,
        ],
    )
    def scatter_kernel(g_hbm, src_hbm, dst_hbm, out_hbm,
                       src_v, dst_v, rows0, rows1, acc, sem0, sem1):
        c = lax.axis_index("c")
        s = lax.axis_index("s")
        wid = c * NS + s

        def fill_zero(i, _):
            def fz(j, _):
                rows0[i, pl.ds(j * 16, 16)] = jnp.zeros((16,), jnp.float32)
                return 0
            lax.fori_loop(0, D // 16, fz, 0)
            return 0

        lax.fori_loop(0, C, fill_zero, 0)

        for (o, k) in offs:
            pltpu.sync_copy(rows0.at[pl.ds(0, k)],
                            acc.at[pl.ds(s * rpt + o, k)])
        plsc.subcore_barrier()

        def gather_start(i, buf, sem):
            pltpu.async_copy(g_hbm.at[src_v.at[i]], buf, sem)

        def gather_wait(i, buf, sem):
            pltpu.make_async_copy(g_hbm.at[src_v.at[i]], buf, sem).wait()

        def scatter(i, buf):
            pltpu.sync_copy(buf, acc.at[dst_v.at[i]], add=True)

        for p in range(phases):
            pltpu.sync_copy(src_hbm.at[wid, pl.ds(p * npp, npp)], src_v)
            pltpu.sync_copy(dst_hbm.at[wid, pl.ds(p * npp, npp)], dst_v)
            gather_start(0, rows0, sem0)

            def body(k2, _):
                i0 = 2 * k2
                i1 = i0 + 1
                gather_start(i1, rows1, sem1)
                gather_wait(i0, rows0, sem0)
                scatter(i0, rows0)

                @pl.when(i0 + 2 < npp)
                def _():
                    gather_start(i0 + 2, rows0, sem0)

                gather_wait(i1, rows1, sem1)
                scatter(i1, rows1)
                return 0

            if npp == 1:
                gather_wait(0, rows0, sem0)
                scatter(0, rows0)
            else:
                lax.fori_loop(0, npp // 2, body, 0)
        plsc.subcore_barrier()

        for (o, k) in offs:
            pltpu.sync_copy(acc.at[pl.ds(s * rpt + o, k)],
                            rows0.at[pl.ds(0, k)])
            pltpu.sync_copy(rows0.at[pl.ds(0, k)],
                            out_hbm.at[c, pl.ds(s * rpt + o, k)])

    return scatter_kernel


_R = 2000


def _dinv_from_deg(deg_ref):
    deg = deg_ref[0, :, 0:1] + deg_ref[1, :, 0:1] + 1.0
    return lax.rsqrt(deg)


def _scale_mm_body(x_ref, w_ref, deg_ref, o_ref):
    o_ref[...] = jnp.dot(x_ref[...], w_ref[...],
                         preferred_element_type=jnp.float32) * _dinv_from_deg(deg_ref)


def _tc_scale_mm(x, w, degp):
    n, d = x.shape
    r = _R if n % _R == 0 else 8
    return pl.pallas_call(
        _scale_mm_body,
        grid=(n // r,),
        in_specs=[
            pl.BlockSpec((r, d), lambda i: (i, 0)),
            pl.BlockSpec((d, w.shape[1]), lambda i: (0, 0)),
            pl.BlockSpec((NC, r, DEG_W), lambda i: (0, i, 0)),
        ],
        out_specs=pl.BlockSpec((r, w.shape[1]), lambda i: (i, 0)),
        out_shape=jax.ShapeDtypeStruct((n, w.shape[1]), jnp.float32),
    )(x, w, degp)


def _layer_body(s_ref, g_ref, deg_ref, b_ref, w_ref, o_ref):
    dinv = _dinv_from_deg(deg_ref)
    tot = s_ref[0] + s_ref[1] + g_ref[...]
    h = jnp.maximum(dinv * tot + b_ref[...], 0.0)
    o_ref[...] = jnp.dot(h, w_ref[...],
                         preferred_element_type=jnp.float32) * dinv


def _tc_layer(S, g, degp, b, w):
    n, d = g.shape
    r = _R if n % _R == 0 else 8
    return pl.pallas_call(
        _layer_body,
        grid=(n // r,),
        in_specs=[
            pl.BlockSpec((NC, r, d), lambda i: (0, i, 0)),
            pl.BlockSpec((r, d), lambda i: (i, 0)),
            pl.BlockSpec((NC, r, DEG_W), lambda i: (0, i, 0)),
            pl.BlockSpec((1, d), lambda i: (0, 0)),
            pl.BlockSpec((d, d), lambda i: (0, 0)),
        ],
        out_specs=pl.BlockSpec((r, d), lambda i: (i, 0)),
        out_shape=jax.ShapeDtypeStruct((n, d), jnp.float32),
    )(S, g, degp, b, w)


def _final_body(s_ref, g_ref, deg_ref, b_ref, w_ref, b3_ref, o_ref):
    dinv = _dinv_from_deg(deg_ref)
    tot = s_ref[0] + s_ref[1] + g_ref[...]
    h = jnp.maximum(dinv * tot + b_ref[...], 0.0)
    res = jnp.dot(h, w_ref[...],
                  preferred_element_type=jnp.float32) + b3_ref[...]
    o_ref[...] = res[:, :o_ref.shape[1]]


def _tc_final(S, g, degp, b, w_pad, b3_pad, n_classes):
    n, d = g.shape
    r = _R if n % _R == 0 else 8
    dp = w_pad.shape[1]
    return pl.pallas_call(
        _final_body,
        grid=(n // r,),
        in_specs=[
            pl.BlockSpec((NC, r, d), lambda i: (0, i, 0)),
            pl.BlockSpec((r, d), lambda i: (i, 0)),
            pl.BlockSpec((NC, r, DEG_W), lambda i: (0, i, 0)),
            pl.BlockSpec((1, d), lambda i: (0, 0)),
            pl.BlockSpec((d, dp), lambda i: (0, 0)),
            pl.BlockSpec((1, dp), lambda i: (0, 0)),
        ],
        out_specs=pl.BlockSpec((r, n_classes), lambda i: (i, 0)),
        out_shape=jax.ShapeDtypeStruct((n, n_classes), jnp.float32),
    )(S, g, degp, b, w_pad, b3_pad)


def kernel(x, edge_index, W1, b1, W2, b2, W3, b3):
    n, d = x.shape
    e = edge_index.shape[1]

    n_classes = W3.shape[1]
    w3_pad = jnp.pad(W3.astype(jnp.float32), ((0, 0), (0, d - n_classes)))
    b3_pad = jnp.pad(b3.astype(jnp.float32), (0, d - n_classes)).reshape(1, d)
    b1r = b1.reshape(1, d)
    b2r = b2.reshape(1, d)

    quant = NW * _C
    n_chunks = -(-e // quant)
    if n_chunks > 1 and n_chunks % 4:
        n_chunks += 4 - n_chunks % 4
    e_pad = n_chunks * quant
    edges = edge_index.astype(jnp.int32)
    if e_pad != e:
        n_trash = _pad_rows(n) - n
        assert n_trash > 0, "no trash row available for edge padding"
        pad = e_pad - e
        pad_block = jnp.stack([jnp.arange(pad, dtype=jnp.int32) % n,
                               n + jnp.arange(pad, dtype=jnp.int32) % n_trash])
        edges = jnp.concatenate([edges, pad_block], axis=1)
    idx3 = edges.reshape(2, NW, n_chunks, _C)
    src3 = idx3[0]
    dst3 = idx3[1]

    deg_fn = _make_deg_kernel(n_chunks, n)
    scat_fn = _make_scatter_kernel(n_chunks, n, d)

    degp = deg_fn(dst3)
    g1 = _tc_scale_mm(x, W1, degp)
    S1 = scat_fn(g1, src3, dst3)
    g2 = _tc_layer(S1, g1, degp, b1r, W2)
    S2 = scat_fn(g2, src3, dst3)
    return _tc_final(S2, g2, degp, b2r, w3_pad, b3_pad, n_classes)


# speedup vs baseline: 1.2537x; 1.0044x over previous
import functools

import jax
import jax.numpy as jnp
from jax import lax
from jax.experimental import pallas as pl
from jax.experimental.pallas import tpu as pltpu
from jax.experimental.pallas import tpu_sc as plsc

NC = 2
NS = 16
NW = NC * NS

DEG_W = 16


_C = 128


def _pick_rows_chunk(rows_per_tile):
    for c in range(128, 0, -1):
        if rows_per_tile % c == 0 and c % 8 == 0:
            return c
    raise ValueError("bad row count")


def _pad_rows(N):
    q = NS * 8
    return ((N + q - 1) // q) * q


def _strip_chunks(rpt):
    offs = []
    o = 0
    while o < rpt:
        k = min(_C, rpt - o)
        offs.append((o, k))
        o += k
    return offs


@functools.lru_cache(maxsize=None)
def _make_deg_kernel(n_chunks, N):
    C = _C
    Np = _pad_rows(N)
    rpt = Np // NS
    offs = _strip_chunks(rpt)
    mesh = plsc.VectorSubcoreMesh(core_axis_name="c", subcore_axis_name="s", num_cores=NC, num_subcores=NS)

    @functools.partial(
        pl.kernel,
        out_type=jax.ShapeDtypeStruct((NC, Np, DEG_W), jnp.float32),
        mesh=mesh,
        compiler_params=pltpu.CompilerParams(use_tc_tiling_on_sc=False),
        scratch_types=[
            pltpu.VMEM((n_chunks, C), jnp.int32),
            pltpu.VMEM((C, DEG_W), jnp.float32),
            pltpu.VMEM((C, DEG_W), jnp.float32),
            pltpu.VMEM_SHARED((Np, DEG_W), jnp.float32),
            pltpu.SemaphoreType.DMA,
        ],
    )
    def deg_kernel(dst_hbm, out_hbm, dst_v, ones_v, bounce_v, acc, sem):
        c = lax.axis_index("c")
        s = lax.axis_index("s")
        wid = c * NS + s

        pltpu.sync_copy(dst_hbm.at[wid], dst_v)

        def fill_ones(i, _):
            def fo(j, _):
                ones_v[i, pl.ds(j * 16, 16)] = jnp.ones((16,), jnp.float32)
                return 0
            lax.fori_loop(0, DEG_W // 16, fo, 0)
            return 0

        lax.fori_loop(0, C, fill_ones, 0)

        def fill_zero(i, _):
            def fz(j, _):
                bounce_v[i, pl.ds(j * 16, 16)] = jnp.zeros((16,), jnp.float32)
                return 0
            lax.fori_loop(0, DEG_W // 16, fz, 0)
            return 0

        lax.fori_loop(0, C, fill_zero, 0)

        for (o, k) in offs:
            pltpu.sync_copy(bounce_v.at[pl.ds(0, k)],
                            acc.at[pl.ds(s * rpt + o, k)])
        plsc.subcore_barrier()

        def chunk(i, _):
            pltpu.async_copy(ones_v, acc.at[dst_v.at[i]], sem, add=True)
            return 0

        lax.fori_loop(0, n_chunks, chunk, 0)

        def drain(i, _):
            pltpu.make_async_copy(ones_v, acc.at[dst_v.at[0]], sem).wait()
            return 0

        lax.fori_loop(0, n_chunks, drain, 0)
        plsc.subcore_barrier()

        pltpu.sync_copy(acc.at[pl.ds(s * rpt, rpt)],
                        out_hbm.at[c, pl.ds(s * rpt, rpt)])

    return deg_kernel


@functools.lru_cache(maxsize=None)
def _make_scatter_kernel(n_chunks, N, D):
    C = _C
    Np = _pad_rows(N)
    rpt = Np // NS
    offs = _strip_chunks(rpt)
    phases = 2 if n_chunks > 1 else 1
    npp = n_chunks // phases
    assert npp * phases == n_chunks and (npp % 2 == 0 or npp == 1)
    mesh = plsc.VectorSubcoreMesh(core_axis_name="c", subcore_axis_name="s", num_cores=NC, num_subcores=NS)

    @functools.partial(
        pl.kernel,
        out_type=jax.ShapeDtypeStruct((NC, Np, D), jnp.float32),
        mesh=mesh,
        scratch_types=[
            pltpu.VMEM((npp, C), jnp.int32),
            pltpu.VMEM((npp, C), jnp.int32),
            pltpu.VMEM((C, D), jnp.float32),
            pltpu.VMEM((C, D), jnp.float32),
            pltpu.VMEM_SHARED((Np, D), jnp.float32),
            pltpu.SemaphoreType.DMA,
            pltpu.SemaphoreType.DMA,
        ],
    )
    def scatter_kernel(g_hbm, src_hbm, dst_hbm, out_hbm,
                       src_v, dst_v, rows0, rows1, acc, sem0, sem1):
        c = lax.axis_index("c")
        s = lax.axis_index("s")
        wid = c * NS + s

        def fill_zero(i, _):
            def fz(j, _):
                rows0[i, pl.ds(j * 16, 16)] = jnp.zeros((16,), jnp.float32)
                return 0
            lax.fori_loop(0, D // 16, fz, 0)
            return 0

        lax.fori_loop(0, C, fill_zero, 0)

        for (o, k) in offs:
            pltpu.sync_copy(rows0.at[pl.ds(0, k)],
                            acc.at[pl.ds(s * rpt + o, k)])
        plsc.subcore_barrier()

        def gather_start(i, buf, sem):
            pltpu.async_copy(g_hbm.at[src_v.at[i]], buf, sem)

        def gather_wait(i, buf, sem):
            pltpu.make_async_copy(g_hbm.at[src_v.at[i]], buf, sem).wait()

        def scatter(i, buf):
            pltpu.sync_copy(buf, acc.at[dst_v.at[i]], add=True)

        for p in range(phases):
            pltpu.sync_copy(src_hbm.at[wid, pl.ds(p * npp, npp)], src_v)
            pltpu.sync_copy(dst_hbm.at[wid, pl.ds(p * npp, npp)], dst_v)
            gather_start(0, rows0, sem0)

            def body(k2, _):
                i0 = 2 * k2
                i1 = i0 + 1
                gather_start(i1, rows1, sem1)
                gather_wait(i0, rows0, sem0)
                scatter(i0, rows0)

                @pl.when(i0 + 2 < npp)
                def _():
                    gather_start(i0 + 2, rows0, sem0)

                gather_wait(i1, rows1, sem1)
                scatter(i1, rows1)
                return 0

            if npp == 1:
                gather_wait(0, rows0, sem0)
                scatter(0, rows0)
            else:
                lax.fori_loop(0, npp // 2, body, 0)
        plsc.subcore_barrier()

        pltpu.sync_copy(acc.at[pl.ds(s * rpt, rpt)],
                        out_hbm.at[c, pl.ds(s * rpt, rpt)])

    return scatter_kernel


_R = 2000


def _dinv_from_deg(deg_ref):
    deg = deg_ref[0, :, 0:1] + deg_ref[1, :, 0:1] + 1.0
    return lax.rsqrt(deg)


def _scale_mm_body(x_ref, w_ref, deg_ref, o_ref):
    o_ref[...] = jnp.dot(x_ref[...], w_ref[...],
                         preferred_element_type=jnp.float32) * _dinv_from_deg(deg_ref)


def _tc_scale_mm(x, w, degp):
    n, d = x.shape
    r = _R if n % _R == 0 else 8
    return pl.pallas_call(
        _scale_mm_body,
        grid=(n // r,),
        in_specs=[
            pl.BlockSpec((r, d), lambda i: (i, 0)),
            pl.BlockSpec((d, w.shape[1]), lambda i: (0, 0)),
            pl.BlockSpec((NC, r, DEG_W), lambda i: (0, i, 0)),
        ],
        out_specs=pl.BlockSpec((r, w.shape[1]), lambda i: (i, 0)),
        out_shape=jax.ShapeDtypeStruct((n, w.shape[1]), jnp.float32),
    )(x, w, degp)


def _layer_body(s_ref, g_ref, deg_ref, b_ref, w_ref, o_ref):
    dinv = _dinv_from_deg(deg_ref)
    tot = s_ref[0] + s_ref[1] + g_ref[...]
    h = jnp.maximum(dinv * tot + b_ref[...], 0.0)
    o_ref[...] = jnp.dot(h, w_ref[...],
                         preferred_element_type=jnp.float32) * dinv


def _tc_layer(S, g, degp, b, w):
    n, d = g.shape
    r = _R if n % _R == 0 else 8
    return pl.pallas_call(
        _layer_body,
        grid=(n // r,),
        in_specs=[
            pl.BlockSpec((NC, r, d), lambda i: (0, i, 0)),
            pl.BlockSpec((r, d), lambda i: (i, 0)),
            pl.BlockSpec((NC, r, DEG_W), lambda i: (0, i, 0)),
            pl.BlockSpec((1, d), lambda i: (0, 0)),
            pl.BlockSpec((d, d), lambda i: (0, 0)),
        ],
        out_specs=pl.BlockSpec((r, d), lambda i: (i, 0)),
        out_shape=jax.ShapeDtypeStruct((n, d), jnp.float32),
    )(S, g, degp, b, w)


def _final_body(s_ref, g_ref, deg_ref, b_ref, w_ref, b3_ref, o_ref):
    dinv = _dinv_from_deg(deg_ref)
    tot = s_ref[0] + s_ref[1] + g_ref[...]
    h = jnp.maximum(dinv * tot + b_ref[...], 0.0)
    res = jnp.dot(h, w_ref[...],
                  preferred_element_type=jnp.float32) + b3_ref[...]
    o_ref[...] = res[:, :o_ref.shape[1]]


def _tc_final(S, g, degp, b, w_pad, b3_pad, n_classes):
    n, d = g.shape
    r = _R if n % _R == 0 else 8
    dp = w_pad.shape[1]
    return pl.pallas_call(
        _final_body,
        grid=(n // r,),
        in_specs=[
            pl.BlockSpec((NC, r, d), lambda i: (0, i, 0)),
            pl.BlockSpec((r, d), lambda i: (i, 0)),
            pl.BlockSpec((NC, r, DEG_W), lambda i: (0, i, 0)),
            pl.BlockSpec((1, d), lambda i: (0, 0)),
            pl.BlockSpec((d, dp), lambda i: (0, 0)),
            pl.BlockSpec((1, dp), lambda i: (0, 0)),
        ],
        out_specs=pl.BlockSpec((r, n_classes), lambda i: (i, 0)),
        out_shape=jax.ShapeDtypeStruct((n, n_classes), jnp.float32),
    )(S, g, degp, b, w_pad, b3_pad)


def kernel(x, edge_index, W1, b1, W2, b2, W3, b3):
    n, d = x.shape
    e = edge_index.shape[1]

    n_classes = W3.shape[1]
    w3_pad = jnp.pad(W3.astype(jnp.float32), ((0, 0), (0, d - n_classes)))
    b3_pad = jnp.pad(b3.astype(jnp.float32), (0, d - n_classes)).reshape(1, d)
    b1r = b1.reshape(1, d)
    b2r = b2.reshape(1, d)

    quant = NW * _C
    n_chunks = -(-e // quant)
    if n_chunks > 1 and n_chunks % 4:
        n_chunks += 4 - n_chunks % 4
    e_pad = n_chunks * quant
    edges = edge_index.astype(jnp.int32)
    if e_pad != e:
        n_trash = _pad_rows(n) - n
        assert n_trash > 0, "no trash row available for edge padding"
        pad = e_pad - e
        pad_block = jnp.stack([jnp.arange(pad, dtype=jnp.int32) % n,
                               n + jnp.arange(pad, dtype=jnp.int32) % n_trash])
        edges = jnp.concatenate([edges, pad_block], axis=1)
    idx3 = edges.reshape(2, NW, n_chunks, _C)
    src3 = idx3[0]
    dst3 = idx3[1]

    deg_fn = _make_deg_kernel(n_chunks, n)
    scat_fn = _make_scatter_kernel(n_chunks, n, d)

    degp = deg_fn(dst3)
    g1 = _tc_scale_mm(x, W1, degp)
    S1 = scat_fn(g1, src3, dst3)
    g2 = _tc_layer(S1, g1, degp, b1r, W2)
    S2 = scat_fn(g2, src3, dst3)
    return _tc_final(S2, g2, degp, b2r, w3_pad, b3_pad, n_classes)
